# baseline (speedup 1.0000x reference)
import functools

import jax
import jax.numpy as jnp
from jax import lax
from jax.experimental import pallas as pl
from jax.experimental.pallas import tpu as pltpu
from jax.experimental.pallas import tpu_sc as plsc

_N = 10000
_D = 128
_K = 4
_DEPTH = 3
_POOL = 0.5
_NC = 2
_NS = 16
_NW = _NC * _NS
_C = 512


def _loops(edge_index, num_nodes):
    loops = jnp.arange(num_nodes, dtype=edge_index.dtype)
    return jnp.concatenate([edge_index, jnp.stack([loops, loops])], axis=1)


def _mm_body(x_ref, w_ref, a2_ref, h0_ref, h1_ref, h2_ref, h3_ref, sd_ref):
    x = x_ref[...]
    w = w_ref[...]
    a2 = a2_ref[...]
    h_refs = (h0_ref, h1_ref, h2_ref, h3_ref)
    cols = []
    for k in range(_K):
        h = jnp.dot(x, w[k], preferred_element_type=jnp.float32)
        h_refs[k][...] = h
        cols.append(jnp.dot(h, a2[k], preferred_element_type=jnp.float32))
    sd_ref[...] = jnp.concatenate(cols, axis=1)


def _dense_stage(x, W, a_src, a_dst):
    blk = 2000
    a2 = jnp.stack([jnp.stack([a_src[k], a_dst[k]], axis=1) for k in range(_K)])
    outs = pl.pallas_call(
        _mm_body,
        grid=(_N // blk,),
        in_specs=[
            pl.BlockSpec((blk, _D), lambda i: (i, 0)),
            pl.BlockSpec((_K, _D, _D), lambda i: (0, 0, 0)),
            pl.BlockSpec((_K, _D, 2), lambda i: (0, 0, 0)),
        ],
        out_specs=[pl.BlockSpec((blk, _D), lambda i: (i, 0))] * _K
        + [pl.BlockSpec((blk, 2 * _K), lambda i: (i, 0))],
        out_shape=[jax.ShapeDtypeStruct((_N, _D), jnp.float32)] * _K
        + [jax.ShapeDtypeStruct((_N, 2 * _K), jnp.float32)],
    )(x, W, a2)
    hs, sd = outs[:_K], outs[_K]
    return hs, sd


def _pad_len(E):
    per_w = ((E + _NW - 1) // _NW + _C - 1) // _C * _C
    return per_w, _NW * per_w


def _make_gather8(E):
    per_w, Ep = _pad_len(E)
    mesh = plsc.VectorSubcoreMesh(core_axis_name="c", subcore_axis_name="s")
    n_chunks = per_w // _C

    @functools.partial(
        pl.kernel,
        mesh=mesh,
        compiler_params=pltpu.CompilerParams(needs_layout_passes=False),
        out_type=[jax.ShapeDtypeStruct((Ep,), jnp.float32)] * (2 * _K),
        scratch_types=[pltpu.VMEM((_C,), jnp.int32)] * 2
        + [pltpu.VMEM((_C,), jnp.float32)] * (2 * _K)
        + [pltpu.VMEM((_N * 2 * _K,), jnp.float32)],
    )
    def gather8(src_hbm, dst_hbm, sd_hbm, o_s0, o_s1, o_s2, o_s3,
                o_d0, o_d1, o_d2, o_d3, src_v, dst_v,
                b_s0, b_s1, b_s2, b_s3, b_d0, b_d1, b_d2, b_d3, tbl_v):
        wid = lax.axis_index("s") * _NC + lax.axis_index("c")
        pltpu.sync_copy(sd_hbm, tbl_v)
        outs = (o_s0, o_s1, o_s2, o_s3, o_d0, o_d1, o_d2, o_d3)
        bufs = (b_s0, b_s1, b_s2, b_s3, b_d0, b_d1, b_d2, b_d3)
        base = wid * per_w
        for i in range(n_chunks):
            off = base + i * _C
            pltpu.sync_copy(src_hbm.at[pl.ds(off, _C)], src_v)
            pltpu.sync_copy(dst_hbm.at[pl.ds(off, _C)], dst_v)

            def body(g, carry):
                s16 = src_v[pl.ds(g * 16, 16)]
                d16 = dst_v[pl.ds(g * 16, 16)]
                for k in range(_K):
                    bufs[k][pl.ds(g * 16, 16)] = plsc.load_gather(
                        tbl_v, [s16 * (2 * _K) + (2 * k)])
                    bufs[_K + k][pl.ds(g * 16, 16)] = plsc.load_gather(
                        tbl_v, [d16 * (2 * _K) + (2 * k + 1)])
                return carry

            lax.fori_loop(0, _C // 16, body, 0)
            for b, o in zip(bufs, outs):
                pltpu.sync_copy(b, o.at[pl.ds(off, _C)])

    return gather8


def _make_gather4(E):
    per_w, Ep = _pad_len(E)
    mesh = plsc.VectorSubcoreMesh(core_axis_name="c", subcore_axis_name="s")
    n_chunks = per_w // _C

    @functools.partial(
        pl.kernel,
        mesh=mesh,
        compiler_params=pltpu.CompilerParams(needs_layout_passes=False),
        out_type=[jax.ShapeDtypeStruct((Ep,), jnp.float32)] * _K,
        scratch_types=[pltpu.VMEM((_C,), jnp.int32)]
        + [pltpu.VMEM((_C,), jnp.float32)] * _K
        + [pltpu.VMEM((_N * _K,), jnp.float32)],
    )
    def gather4(dst_hbm, tbl_hbm, o0, o1, o2, o3, dst_v, b0, b1, b2, b3, tbl_v):
        wid = lax.axis_index("s") * _NC + lax.axis_index("c")
        pltpu.sync_copy(tbl_hbm, tbl_v)
        outs = (o0, o1, o2, o3)
        bufs = (b0, b1, b2, b3)
        base = wid * per_w
        for i in range(n_chunks):
            off = base + i * _C
            pltpu.sync_copy(dst_hbm.at[pl.ds(off, _C)], dst_v)

            def body(g, carry):
                d16 = dst_v[pl.ds(g * 16, 16)]
                for k in range(_K):
                    bufs[k][pl.ds(g * 16, 16)] = plsc.load_gather(
                        tbl_v, [d16 * _K + k])
                return carry

            lax.fori_loop(0, _C // 16, body, 0)
            for b, o in zip(bufs, outs):
                pltpu.sync_copy(b, o.at[pl.ds(off, _C)])

    return gather4


def _layer(x, ei, W, a_src, a_dst):
    E = ei.shape[1]
    src, dst = ei[0], ei[1]
    per_w, Ep = _pad_len(E)
    pad = jnp.zeros((Ep - E,), jnp.int32)
    srcp = jnp.concatenate([src, pad])
    dstp = jnp.concatenate([dst, pad])

    hs, sd = _dense_stage(x, W, a_src, a_dst)
    sdflat = sd.reshape(-1)

    g8 = _make_gather8(E)(srcp, dstp, sdflat)
    es = []
    for k in range(_K):
        es.append(jax.nn.leaky_relu(g8[k][:E] + g8[_K + k][:E], 0.2))

    emaxs = []
    for k in range(_K):
        m = jax.ops.segment_max(es[k], dst, num_segments=_N)
        emaxs.append(jnp.where(jnp.isfinite(m), m, 0.0))
    emaxflat = jnp.stack(emaxs, axis=1).reshape(-1)

    gm = _make_gather4(E)(dstp, emaxflat)
    exs = [jnp.exp(es[k] - gm[k][:E]) for k in range(_K)]

    dens = [jax.ops.segment_sum(exs[k], dst, num_segments=_N) for k in range(_K)]
    denflat = jnp.stack(dens, axis=1).reshape(-1)

    gn = _make_gather4(E)(dstp, denflat)
    attns = [exs[k] / (gn[k][:E] + 1e-16) for k in range(_K)]

    outs = [jax.ops.segment_sum(attns[k][:, None] * hs[k][src], dst,
                                num_segments=_N) for k in range(_K)]
    out = sum(outs) / float(_K)
    attn_mean = sum(attns) / float(_K)
    return out, attn_mean


def _edge_reduction(edge_index, attn, rate):
    E = attn.shape[0]
    kk = max(int(E * rate), 1)
    if E % 16 == 0 and E >= 262144:
        iota = lax.iota(jnp.int32, E)
        _, idx = lax.sort_key_val(-attn, iota, is_stable=True)
        idx = idx[:kk]
    else:
        _, idx = jax.lax.top_k(attn, kk)
    return edge_index[:, idx]


def kernel(x, edge_index, W1, W2, W3, As1, Ad1, As2, Ad2, As3, Ad3):
    params = [(W1, As1, Ad1), (W2, As2, Ad2), (W3, As3, Ad3)]
    edge_list = []
    ei = _loops(edge_index, x.shape[0])
    for i in range(_DEPTH):
        edge_list.append(ei)
        x, attn = _layer(x, ei, params[i][0], params[i][1], params[i][2])
        x = jax.nn.leaky_relu(x, 0.01)
        x = x / jnp.maximum(jnp.linalg.norm(x, axis=0, keepdims=True), 1e-12)
        ei = _edge_reduction(ei, attn, _POOL)
        ei = _loops(ei, x.shape[0])
    return (x, ei) + tuple(edge_list)


# speedup vs baseline: 1.0017x; 1.0017x over previous
import functools

import jax
import jax.numpy as jnp
from jax import lax
from jax.experimental import pallas as pl
from jax.experimental.pallas import tpu as pltpu
from jax.experimental.pallas import tpu_sc as plsc

_N = 10000
_D = 128
_K = 4
_DEPTH = 3
_POOL = 0.5
_NC = 2
_NS = 16
_NW = _NC * _NS
_C = 2048


def _loops(edge_index, num_nodes):
    loops = jnp.arange(num_nodes, dtype=edge_index.dtype)
    return jnp.concatenate([edge_index, jnp.stack([loops, loops])], axis=1)


def _mm_body(x_ref, w_ref, a2_ref, h0_ref, h1_ref, h2_ref, h3_ref, sd_ref):
    x = x_ref[...]
    w = w_ref[...]
    a2 = a2_ref[...]
    h_refs = (h0_ref, h1_ref, h2_ref, h3_ref)
    cols = []
    for k in range(_K):
        h = jnp.dot(x, w[k], preferred_element_type=jnp.float32)
        h_refs[k][...] = h
        cols.append(jnp.dot(h, a2[k], preferred_element_type=jnp.float32))
    sd_ref[...] = jnp.concatenate(cols, axis=1)


def _dense_stage(x, W, a_src, a_dst):
    blk = 2000
    a2 = jnp.stack([jnp.stack([a_src[k], a_dst[k]], axis=1) for k in range(_K)])
    outs = pl.pallas_call(
        _mm_body,
        grid=(_N // blk,),
        in_specs=[
            pl.BlockSpec((blk, _D), lambda i: (i, 0)),
            pl.BlockSpec((_K, _D, _D), lambda i: (0, 0, 0)),
            pl.BlockSpec((_K, _D, 2), lambda i: (0, 0, 0)),
        ],
        out_specs=[pl.BlockSpec((blk, _D), lambda i: (i, 0))] * _K
        + [pl.BlockSpec((blk, 2 * _K), lambda i: (i, 0))],
        out_shape=[jax.ShapeDtypeStruct((_N, _D), jnp.float32)] * _K
        + [jax.ShapeDtypeStruct((_N, 2 * _K), jnp.float32)],
    )(x, W, a2)
    hs, sd = outs[:_K], outs[_K]
    return hs, sd


def _pad_len(E):
    per_w = ((E + _NW - 1) // _NW + _C - 1) // _C * _C
    return per_w, _NW * per_w


def _make_gather8(E):
    per_w, Ep = _pad_len(E)
    mesh = plsc.VectorSubcoreMesh(core_axis_name="c", subcore_axis_name="s")
    n_chunks = per_w // _C

    @functools.partial(
        pl.kernel,
        mesh=mesh,
        compiler_params=pltpu.CompilerParams(needs_layout_passes=False),
        out_type=[jax.ShapeDtypeStruct((Ep,), jnp.float32)] * (2 * _K),
        scratch_types=[pltpu.VMEM((_C,), jnp.int32)] * 2
        + [pltpu.VMEM((_C,), jnp.float32)] * (2 * _K)
        + [pltpu.VMEM((_N * 2 * _K,), jnp.float32)],
    )
    def gather8(src_hbm, dst_hbm, sd_hbm, o_s0, o_s1, o_s2, o_s3,
                o_d0, o_d1, o_d2, o_d3, src_v, dst_v,
                b_s0, b_s1, b_s2, b_s3, b_d0, b_d1, b_d2, b_d3, tbl_v):
        wid = lax.axis_index("s") * _NC + lax.axis_index("c")
        pltpu.sync_copy(sd_hbm, tbl_v)
        outs = (o_s0, o_s1, o_s2, o_s3, o_d0, o_d1, o_d2, o_d3)
        bufs = (b_s0, b_s1, b_s2, b_s3, b_d0, b_d1, b_d2, b_d3)
        base = wid * per_w
        for i in range(n_chunks):
            off = base + i * _C
            pltpu.sync_copy(src_hbm.at[pl.ds(off, _C)], src_v)
            pltpu.sync_copy(dst_hbm.at[pl.ds(off, _C)], dst_v)

            def body(g, carry):
                s16 = src_v[pl.ds(g * 16, 16)]
                d16 = dst_v[pl.ds(g * 16, 16)]
                for k in range(_K):
                    bufs[k][pl.ds(g * 16, 16)] = plsc.load_gather(
                        tbl_v, [s16 * (2 * _K) + (2 * k)])
                    bufs[_K + k][pl.ds(g * 16, 16)] = plsc.load_gather(
                        tbl_v, [d16 * (2 * _K) + (2 * k + 1)])
                return carry

            lax.fori_loop(0, _C // 16, body, 0)
            for b, o in zip(bufs, outs):
                pltpu.sync_copy(b, o.at[pl.ds(off, _C)])

    return gather8


def _make_gather4(E):
    per_w, Ep = _pad_len(E)
    mesh = plsc.VectorSubcoreMesh(core_axis_name="c", subcore_axis_name="s")
    n_chunks = per_w // _C

    @functools.partial(
        pl.kernel,
        mesh=mesh,
        compiler_params=pltpu.CompilerParams(needs_layout_passes=False),
        out_type=[jax.ShapeDtypeStruct((Ep,), jnp.float32)] * _K,
        scratch_types=[pltpu.VMEM((_C,), jnp.int32)]
        + [pltpu.VMEM((_C,), jnp.float32)] * _K
        + [pltpu.VMEM((_N * _K,), jnp.float32)],
    )
    def gather4(dst_hbm, tbl_hbm, o0, o1, o2, o3, dst_v, b0, b1, b2, b3, tbl_v):
        wid = lax.axis_index("s") * _NC + lax.axis_index("c")
        pltpu.sync_copy(tbl_hbm, tbl_v)
        outs = (o0, o1, o2, o3)
        bufs = (b0, b1, b2, b3)
        base = wid * per_w
        for i in range(n_chunks):
            off = base + i * _C
            pltpu.sync_copy(dst_hbm.at[pl.ds(off, _C)], dst_v)

            def body(g, carry):
                d16 = dst_v[pl.ds(g * 16, 16)]
                for k in range(_K):
                    bufs[k][pl.ds(g * 16, 16)] = plsc.load_gather(
                        tbl_v, [d16 * _K + k])
                return carry

            lax.fori_loop(0, _C // 16, body, 0)
            for b, o in zip(bufs, outs):
                pltpu.sync_copy(b, o.at[pl.ds(off, _C)])

    return gather4


def _layer(x, ei, W, a_src, a_dst):
    E = ei.shape[1]
    src, dst = ei[0], ei[1]
    per_w, Ep = _pad_len(E)
    pad = jnp.zeros((Ep - E,), jnp.int32)
    srcp = jnp.concatenate([src, pad])
    dstp = jnp.concatenate([dst, pad])

    hs, sd = _dense_stage(x, W, a_src, a_dst)
    sdflat = sd.reshape(-1)

    g8 = _make_gather8(E)(srcp, dstp, sdflat)
    es = []
    for k in range(_K):
        es.append(jax.nn.leaky_relu(g8[k][:E] + g8[_K + k][:E], 0.2))

    emaxs = []
    for k in range(_K):
        m = jax.ops.segment_max(es[k], dst, num_segments=_N)
        emaxs.append(jnp.where(jnp.isfinite(m), m, 0.0))
    emaxflat = jnp.stack(emaxs, axis=1).reshape(-1)

    gm = _make_gather4(E)(dstp, emaxflat)
    exs = [jnp.exp(es[k] - gm[k][:E]) for k in range(_K)]

    dens = [jax.ops.segment_sum(exs[k], dst, num_segments=_N) for k in range(_K)]
    denflat = jnp.stack(dens, axis=1).reshape(-1)

    gn = _make_gather4(E)(dstp, denflat)
    attns = [exs[k] / (gn[k][:E] + 1e-16) for k in range(_K)]

    outs = [jax.ops.segment_sum(attns[k][:, None] * hs[k][src], dst,
                                num_segments=_N) for k in range(_K)]
    out = sum(outs) / float(_K)
    attn_mean = sum(attns) / float(_K)
    return out, attn_mean


def _edge_reduction(edge_index, attn, rate):
    E = attn.shape[0]
    kk = max(int(E * rate), 1)
    if E % 16 == 0 and E >= 262144:
        iota = lax.iota(jnp.int32, E)
        _, idx = lax.sort_key_val(-attn, iota, is_stable=True)
        idx = idx[:kk]
    else:
        _, idx = jax.lax.top_k(attn, kk)
    return edge_index[:, idx]


def kernel(x, edge_index, W1, W2, W3, As1, Ad1, As2, Ad2, As3, Ad3):
    params = [(W1, As1, Ad1), (W2, As2, Ad2), (W3, As3, Ad3)]
    edge_list = []
    ei = _loops(edge_index, x.shape[0])
    for i in range(_DEPTH):
        edge_list.append(ei)
        x, attn = _layer(x, ei, params[i][0], params[i][1], params[i][2])
        x = jax.nn.leaky_relu(x, 0.01)
        x = x / jnp.maximum(jnp.linalg.norm(x, axis=0, keepdims=True), 1e-12)
        ei = _edge_reduction(ei, attn, _POOL)
        ei = _loops(ei, x.shape[0])
    return (x, ei) + tuple(edge_list)
